# baseline (speedup 1.0000x reference)
import functools

import jax
import jax.numpy as jnp
from jax import lax
from jax.experimental import pallas as pl
from jax.experimental.pallas import tpu as pltpu
from jax.experimental.pallas import tpu_sc as plsc

B = 64
D = 128
K = 16
T = 0.07
N = B * B * K
NC = 2
NS = 16
NW = NC * NS
ROWS_W = N // NW
CHUNK = 128
NCHUNK = ROWS_W // CHUNK


def _sc_gather(table, idx3, half):
    nchunk = NCHUNK // 2
    rows_w = nchunk * CHUNK
    mesh = plsc.VectorSubcoreMesh(core_axis_name="c", subcore_axis_name="s")

    depth = 4
    out_lag = 2

    @functools.partial(
        pl.kernel,
        mesh=mesh,
        out_type=jax.ShapeDtypeStruct((NW * rows_w, D), jnp.float32),
        scratch_types=(
            [pltpu.VMEM((nchunk, CHUNK), jnp.int32)]
            + [pltpu.VMEM((CHUNK, D), jnp.float32)] * depth
            + [pltpu.SemaphoreType.DMA] * (2 * depth)
        ),
    )
    def gather_k(table_hbm, idx_hbm, out_hbm, idx_v, *rest):
        bufs = rest[:depth]
        gsem = rest[depth:2 * depth]
        osem = rest[2 * depth:]
        wid = lax.axis_index("s") * NC + lax.axis_index("c")
        pltpu.sync_copy(
            idx_hbm.at[half * (NW // 2) + wid // 2,
                       pl.ds((wid % 2) * nchunk, nchunk)], idx_v)
        base = wid * rows_w
        gat = [None] * nchunk
        ocp = [None] * nchunk

        def wb(c):
            gat[c].wait()
            ocp[c] = pltpu.async_copy(
                bufs[c % depth],
                out_hbm.at[pl.ds(base + c * CHUNK, CHUNK)],
                osem[c % depth])

        for c in range(nchunk):
            if c >= depth:
                ocp[c - depth].wait()
            gat[c] = pltpu.async_copy(
                table_hbm.at[idx_v.at[c]], bufs[c % depth], gsem[c % depth])
            if c >= out_lag:
                wb(c - out_lag)
        for c in range(nchunk - out_lag, nchunk):
            wb(c)
        for c in range(max(0, nchunk - depth), nchunk):
            if ocp[c] is not None:
                ocp[c].wait()

    return gather_k(table, idx3)


def _nt(x, w):
    return lax.dot_general(x, w, (((1,), (1,)), ((), ())),
                           preferred_element_type=jnp.float32)


def _mm(x, w, b):
    return _nt(x, w) + b


def _stage1_body(s_ref, t_ref, wes, bes, wet, bet, wmtv, bmtv, wmtq,
                 bmtq, wmtsv, bmtsv, wmtsq, bmtsq, wmt, bmt, wmts, bmts, wht,
                 bht, whts, bhts, ht_ref, vb_ref, pos_ref, e_ref):
    e2 = jnp.equal(
        lax.broadcasted_iota(jnp.int32, (G2, P), 0) // K,
        lax.broadcasted_iota(jnp.int32, (G2, P), 1)).astype(jnp.float32)
    e_ref[...] = e2
    e2v = jnp.equal(
        (lax.broadcasted_iota(jnp.int32, (G2, B), 0) // K) % B,
        lax.broadcasted_iota(jnp.int32, (G2, B), 1)).astype(jnp.float32)
    se = _mm(s_ref[...], wes[...], bes[...])
    te = _mm(t_ref[...], wet[...], bet[...])
    m_t_v = _mm(te, wmtv[...], bmtv[...])
    m_t_q = _mm(te, wmtq[...], bmtq[...])
    v = _mm(te, wmtsv[...], bmtsv[...])
    q_pos = _mm(se, wmtsq[...], bmtsq[...])
    dt = jax.nn.relu(m_t_v[None, :, :] - m_t_q[:, None, :]).reshape(B * B, D)
    rt = _mm(dt, wmt[...], bmt[...])
    ht = _mm(rt, wht[...], bht[...])
    ht = ht / jnp.sqrt(jnp.sum(ht * ht, axis=-1, keepdims=True))
    ht_ref[...] = ht
    vb_ref[...] = jnp.dot(e2v, v, preferred_element_type=jnp.float32)
    dp = jax.nn.relu(v[None, :, :] - q_pos[:, None, :]).reshape(B * B, D)
    rp = _mm(dp, wmts[...], bmts[...])
    hp = _mm(rp, whts[...], bhts[...])
    hp = hp / jnp.sqrt(jnp.sum(hp * hp, axis=-1, keepdims=True))
    pos_ref[...] = jnp.exp(
        (jnp.sum(ht * hp, axis=-1, keepdims=True) - 1.0) / T)


BK = B * K
G2 = 4 * BK
P = G2 // K


def _stage2_body(neg_hbm, ht_ref, vb_ref, e_ref, wmtsq, bmtsq, wmts, bmts,
                 whts, bhts, out_ref, nbuf, sem):
    a = pl.program_id(0)
    na = pl.num_programs(0)
    slot = lax.rem(a, 2)
    nxt = lax.rem(a + 1, 2)

    @pl.when(a == 0)
    def _():
        pltpu.make_async_copy(
            neg_hbm.at[pl.ds(0, G2)], nbuf.at[0], sem.at[0]).start()

    @pl.when(a + 1 < na)
    def _():
        pltpu.make_async_copy(
            neg_hbm.at[pl.ds((a + 1) * G2, G2)], nbuf.at[nxt],
            sem.at[nxt]).start()

    pltpu.make_async_copy(
        neg_hbm.at[pl.ds(a * G2, G2)], nbuf.at[slot], sem.at[slot]).wait()
    q = _mm(nbuf[slot], wmtsq[...], bmtsq[...])
    r = _mm(jax.nn.relu(vb_ref[...] - q), wmts[...], bmts[...])
    h = _mm(r, whts[...], bhts[...])
    h = h * lax.rsqrt(jnp.sum(h * h, axis=-1, keepdims=True))
    htb = jnp.dot(e_ref[...], ht_ref[...], preferred_element_type=jnp.float32)
    p = (h * htb).reshape(P, K, D)
    o = jnp.sum(p, axis=-1)
    out_ref[:, :K] = jnp.exp((o - 1.0) / T)


def _full(shape):
    return pl.BlockSpec(shape, lambda a: tuple(0 for _ in shape))


def kernel(s, t, y, idx, memory_s, W_embed_s, b_embed_s, W_embed_t, b_embed_t,
           W_mtv, b_mtv, W_mtq, b_mtq, W_mtsv, b_mtsv, W_mtsq, b_mtsq,
           W_mt, b_mt, W_mts, b_mts, W_ht, b_ht, W_hts, b_hts):
    del y
    idx3 = idx.reshape(-1).astype(jnp.int32).reshape(NW, NCHUNK, CHUNK)
    negs = [_sc_gather(memory_s, idx3, h) for h in range(2)]

    b2 = lambda b: b.reshape(1, -1)

    ht, vb, pos, expand = pl.pallas_call(
        _stage1_body,
        out_shape=(
            jax.ShapeDtypeStruct((B * B, D), jnp.float32),
            jax.ShapeDtypeStruct((G2, D), jnp.float32),
            jax.ShapeDtypeStruct((B * B, 1), jnp.float32),
            jax.ShapeDtypeStruct((G2, P), jnp.float32),
        ),
    )(s, t, W_embed_s, b2(b_embed_s), W_embed_t, b2(b_embed_t),
      W_mtv, b2(b_mtv), W_mtq, b2(b_mtq), W_mtsv, b2(b_mtsv),
      W_mtsq, b2(b_mtsq), W_mt, b2(b_mt), W_mts, b2(b_mts),
      W_ht, b2(b_ht), W_hts, b2(b_hts))

    nsteps = N // 2 // G2
    halves = []
    for h in range(2):
        off = h * nsteps
        halves.append(pl.pallas_call(
            _stage2_body,
            grid=(nsteps,),
            in_specs=[
                pl.BlockSpec(memory_space=pl.ANY),
                pl.BlockSpec((P, D), lambda a, off=off: (a + off, 0)),
                _full((G2, D)),
                _full((G2, P)),
                _full((D, D)), _full((1, D)),
                _full((D, D)), _full((1, D)),
                _full((D, D)), _full((1, D)),
            ],
            out_specs=pl.BlockSpec((P, D), lambda a: (a, 0)),
            out_shape=jax.ShapeDtypeStruct((nsteps * P, D), jnp.float32),
            scratch_shapes=[
                pltpu.VMEM((2, G2, D), jnp.float32),
                pltpu.SemaphoreType.DMA((2,)),
            ],
        )(negs[h], ht, vb, expand, W_mtsq, b2(b_mtsq), W_mts, b2(b_mts),
          W_hts, b2(b_hts)))

    out_neg = jnp.concatenate(halves, axis=0)[:, :K]
    out = jnp.concatenate([pos, out_neg], axis=1)
    return out.reshape(B * B, K + 1, 1)


# speedup vs baseline: 1.0138x; 1.0138x over previous
import functools

import jax
import jax.numpy as jnp
from jax import lax
from jax.experimental import pallas as pl
from jax.experimental.pallas import tpu as pltpu
from jax.experimental.pallas import tpu_sc as plsc

B = 64
D = 128
K = 16
T = 0.07
N = B * B * K
NC = 2
NS = 16
NW = NC * NS
ROWS_W = N // NW
CHUNK = 128
NCHUNK = ROWS_W // CHUNK


def _sc_gather(table, idx3, half):
    nchunk = NCHUNK // 2
    rows_w = nchunk * CHUNK
    mesh = plsc.VectorSubcoreMesh(core_axis_name="c", subcore_axis_name="s")

    depth = 4
    out_lag = 2

    @functools.partial(
        pl.kernel,
        mesh=mesh,
        out_type=jax.ShapeDtypeStruct((NW * rows_w, D), jnp.float32),
        scratch_types=(
            [pltpu.VMEM((nchunk, CHUNK), jnp.int32)]
            + [pltpu.VMEM((CHUNK, D), jnp.float32)] * depth
            + [pltpu.SemaphoreType.DMA] * (2 * depth)
        ),
    )
    def gather_k(table_hbm, idx_hbm, out_hbm, idx_v, *rest):
        bufs = rest[:depth]
        gsem = rest[depth:2 * depth]
        osem = rest[2 * depth:]
        wid = lax.axis_index("s") * NC + lax.axis_index("c")
        pltpu.sync_copy(
            idx_hbm.at[half * (NW // 2) + wid // 2,
                       pl.ds((wid % 2) * nchunk, nchunk)], idx_v)
        base = wid * rows_w
        gat = [None] * nchunk
        ocp = [None] * nchunk

        def wb(c):
            gat[c].wait()
            ocp[c] = pltpu.async_copy(
                bufs[c % depth],
                out_hbm.at[pl.ds(base + c * CHUNK, CHUNK)],
                osem[c % depth])

        for c in range(nchunk):
            if c >= depth:
                ocp[c - depth].wait()
            gat[c] = pltpu.async_copy(
                table_hbm.at[idx_v.at[c]], bufs[c % depth], gsem[c % depth])
            if c >= out_lag:
                wb(c - out_lag)
        for c in range(nchunk - out_lag, nchunk):
            wb(c)
        for c in range(max(0, nchunk - depth), nchunk):
            if ocp[c] is not None:
                ocp[c].wait()

    return gather_k(table, idx3)


def _nt(x, w):
    return lax.dot_general(x, w, (((1,), (1,)), ((), ())),
                           preferred_element_type=jnp.float32)


def _mm(x, w, b):
    return _nt(x, w) + b


def _stage1_body(s_ref, t_ref, wes, bes, wet, bet, wmtv, bmtv, wmtq,
                 bmtq, wmtsv, bmtsv, wmtsq, bmtsq, wmt, bmt, wmts, bmts, wht,
                 bht, whts, bhts, ht_ref, vb_ref, pos_ref, e_ref):
    e2 = jnp.equal(
        lax.broadcasted_iota(jnp.int32, (G2, P), 0) // K,
        lax.broadcasted_iota(jnp.int32, (G2, P), 1)).astype(jnp.float32)
    e_ref[...] = e2
    e2v = jnp.equal(
        (lax.broadcasted_iota(jnp.int32, (G2, B), 0) // K) % B,
        lax.broadcasted_iota(jnp.int32, (G2, B), 1)).astype(jnp.float32)
    se = _mm(s_ref[...], wes[...], bes[...])
    te = _mm(t_ref[...], wet[...], bet[...])
    m_t_v = _mm(te, wmtv[...], bmtv[...])
    m_t_q = _mm(te, wmtq[...], bmtq[...])
    v = _mm(te, wmtsv[...], bmtsv[...])
    q_pos = _mm(se, wmtsq[...], bmtsq[...])
    dt = jax.nn.relu(m_t_v[None, :, :] - m_t_q[:, None, :]).reshape(B * B, D)
    rt = _mm(dt, wmt[...], bmt[...])
    ht = _mm(rt, wht[...], bht[...])
    ht = ht / jnp.sqrt(jnp.sum(ht * ht, axis=-1, keepdims=True))
    ht_ref[...] = ht
    vb_ref[...] = jnp.dot(e2v, v, preferred_element_type=jnp.float32)
    dp = jax.nn.relu(v[None, :, :] - q_pos[:, None, :]).reshape(B * B, D)
    rp = _mm(dp, wmts[...], bmts[...])
    hp = _mm(rp, whts[...], bhts[...])
    hp = hp / jnp.sqrt(jnp.sum(hp * hp, axis=-1, keepdims=True))
    pos_ref[...] = jnp.exp(
        (jnp.sum(ht * hp, axis=-1, keepdims=True) - 1.0) / T)


BK = B * K
G2 = 2 * BK
P = G2 // K


def _stage2_body(neg_hbm, ht_ref, pos_ref, vb_ref, e_ref, wmtsq, bmtsq, wmts,
                 bmts, whts, bhts, out_ref, nbuf, sem):
    a = pl.program_id(0)
    na = pl.num_programs(0)
    slot = lax.rem(a, 2)
    nxt = lax.rem(a + 1, 2)

    @pl.when(a == 0)
    def _():
        pltpu.make_async_copy(
            neg_hbm.at[pl.ds(0, G2)], nbuf.at[0], sem.at[0]).start()

    @pl.when(a + 1 < na)
    def _():
        pltpu.make_async_copy(
            neg_hbm.at[pl.ds((a + 1) * G2, G2)], nbuf.at[nxt],
            sem.at[nxt]).start()

    pltpu.make_async_copy(
        neg_hbm.at[pl.ds(a * G2, G2)], nbuf.at[slot], sem.at[slot]).wait()
    q = _mm(nbuf[slot], wmtsq[...], bmtsq[...])
    r = _mm(jax.nn.relu(vb_ref[...] - q), wmts[...], bmts[...])
    h = _mm(r, whts[...], bhts[...])
    h = h * lax.rsqrt(jnp.sum(h * h, axis=-1, keepdims=True))
    htb = jnp.dot(e_ref[...], ht_ref[...], preferred_element_type=jnp.float32)
    p = (h * htb).reshape(P, K, D)
    o = jnp.sum(p, axis=-1)
    out_ref[:, 0:1] = pos_ref[...]
    out_ref[:, 1:K + 1] = jnp.exp((o - 1.0) / T)


def _full(shape):
    return pl.BlockSpec(shape, lambda a: tuple(0 for _ in shape))


def kernel(s, t, y, idx, memory_s, W_embed_s, b_embed_s, W_embed_t, b_embed_t,
           W_mtv, b_mtv, W_mtq, b_mtq, W_mtsv, b_mtsv, W_mtsq, b_mtsq,
           W_mt, b_mt, W_mts, b_mts, W_ht, b_ht, W_hts, b_hts):
    del y
    idx3 = idx.reshape(-1).astype(jnp.int32).reshape(NW, NCHUNK, CHUNK)
    negs = [_sc_gather(memory_s, idx3, h) for h in range(2)]

    b2 = lambda b: b.reshape(1, -1)

    ht, vb, pos, expand = pl.pallas_call(
        _stage1_body,
        out_shape=(
            jax.ShapeDtypeStruct((B * B, D), jnp.float32),
            jax.ShapeDtypeStruct((G2, D), jnp.float32),
            jax.ShapeDtypeStruct((B * B, 1), jnp.float32),
            jax.ShapeDtypeStruct((G2, P), jnp.float32),
        ),
    )(s, t, W_embed_s, b2(b_embed_s), W_embed_t, b2(b_embed_t),
      W_mtv, b2(b_mtv), W_mtq, b2(b_mtq), W_mtsv, b2(b_mtsv),
      W_mtsq, b2(b_mtsq), W_mt, b2(b_mt), W_mts, b2(b_mts),
      W_ht, b2(b_ht), W_hts, b2(b_hts))

    nsteps = N // 2 // G2
    halves = []
    for h in range(2):
        off = h * nsteps
        halves.append(pl.pallas_call(
            _stage2_body,
            grid=(nsteps,),
            in_specs=[
                pl.BlockSpec(memory_space=pl.ANY),
                pl.BlockSpec((P, D), lambda a, off=off: (a + off, 0)),
                pl.BlockSpec((P, 1), lambda a, off=off: (a + off, 0)),
                _full((G2, D)),
                _full((G2, P)),
                _full((D, D)), _full((1, D)),
                _full((D, D)), _full((1, D)),
                _full((D, D)), _full((1, D)),
            ],
            out_specs=pl.BlockSpec((P, D), lambda a: (a, 0)),
            out_shape=jax.ShapeDtypeStruct((nsteps * P, D), jnp.float32),
            scratch_shapes=[
                pltpu.VMEM((2, G2, D), jnp.float32),
                pltpu.SemaphoreType.DMA((2,)),
            ],
        )(negs[h], ht, pos, vb, expand, W_mtsq, b2(b_mtsq), W_mts, b2(b_mts),
          W_hts, b2(b_hts)))

    out = jnp.concatenate(halves, axis=0)[:, :K + 1]
    return out.reshape(B * B, K + 1, 1)
